# Initial kernel scaffold; baseline (speedup 1.0000x reference)
import functools

import jax
import jax.numpy as jnp
from jax import lax
from jax.experimental import pallas as pl
from jax.experimental.pallas import tpu as pltpu
from jax.experimental.pallas import tpu_sc as plsc

N = 320000
D = 128
S = 64
NC = 2
NS = 16
NW = NC * NS
R = N // NW
B = 400
NB = R // B
L = 16
CH = B // L
G = D // L

_mesh = plsc.VectorSubcoreMesh(
    core_axis_name="c", subcore_axis_name="s", num_cores=NC, num_subcores=NS)


def _worker_id():
    return lax.axis_index("s") * NC + lax.axis_index("c")


def _row_seg(iv, r):
    lane = lax.iota(jnp.int32, L)
    return jnp.sum(jnp.where(lane == r, iv, 0))


@functools.partial(
    pl.kernel,
    out_type=[
        jax.ShapeDtypeStruct((NW, S, D), jnp.float32),
        jax.ShapeDtypeStruct((NW, S, L), jnp.float32),
    ],
    mesh=_mesh,
    scratch_types=[
        pltpu.VMEM((B, D), jnp.float32),
        pltpu.VMEM((CH, L), jnp.int32),
        pltpu.VMEM((S, D), jnp.float32),
        pltpu.VMEM((S, L), jnp.float32),
    ],
)
def _seg_partials(feats_hbm, idx_hbm, sums_hbm, cnts_hbm,
                  rows_v, idxb_v, acc_v, cnt_v):
    wid = _worker_id()
    base = wid * R
    zf = jnp.zeros((L,), jnp.float32)

    def zero_body(s, _):
        for j in range(G):
            acc_v[s, pl.ds(j * L, L)] = zf
        cnt_v[s, :] = zf
        return 0
    lax.fori_loop(0, S, zero_body, 0)

    def block_body(b, _):
        row0 = base + b * B
        pltpu.sync_copy(feats_hbm.at[pl.ds(row0, B), :], rows_v)
        pltpu.sync_copy(idx_hbm.at[pl.ds(row0 // L, CH), :], idxb_v)

        def chunk_body(c, _):
            iv = idxb_v[c, :]
            smin = jnp.min(iv)
            smax = jnp.max(iv)
            r0 = c * L

            @pl.when(smin == smax)
            def _fast():
                for j in range(G):
                    cs = pl.ds(j * L, L)
                    sacc = rows_v[r0, cs]
                    for r in range(1, L):
                        sacc = sacc + rows_v[r0 + r, cs]
                    acc_v[smin, cs] = acc_v[smin, cs] + sacc
                cnt_v[smin, :] = cnt_v[smin, :] + jnp.full((L,), 16.0, jnp.float32)

            @pl.when(smin != smax)
            def _slow():
                def row_body(r, _):
                    sr = _row_seg(iv, r)
                    for j in range(G):
                        cs = pl.ds(j * L, L)
                        acc_v[sr, cs] = acc_v[sr, cs] + rows_v[r0 + r, cs]
                    cnt_v[sr, :] = cnt_v[sr, :] + jnp.full((L,), 1.0, jnp.float32)
                    return 0
                lax.fori_loop(0, L, row_body, 0)

            return 0
        lax.fori_loop(0, CH, chunk_body, 0)
        return 0
    lax.fori_loop(0, NB, block_body, 0)

    pltpu.sync_copy(acc_v, sums_hbm.at[wid])
    pltpu.sync_copy(cnt_v, cnts_hbm.at[wid])


@functools.partial(
    pl.kernel,
    out_type=jax.ShapeDtypeStruct((N, D), jnp.float32),
    mesh=_mesh,
    scratch_types=[
        pltpu.VMEM((B, D), jnp.float32),
        pltpu.VMEM((CH, L), jnp.int32),
        pltpu.VMEM((S, D), jnp.float32),
        pltpu.VMEM((S, D), jnp.float32),
        pltpu.VMEM((S, L), jnp.float32),
        pltpu.VMEM((S, L), jnp.float32),
    ],
)
def _apply(feats_hbm, idx_hbm, sums_hbm, cnts_hbm, out_hbm,
           rows_v, idxb_v, attn_v, stage_v, ctot_v, cstage_v):
    wid = _worker_id()
    base = wid * R
    zf = jnp.zeros((L,), jnp.float32)

    def zero_body(s, _):
        for j in range(G):
            attn_v[s, pl.ds(j * L, L)] = zf
        ctot_v[s, :] = zf
        return 0
    lax.fori_loop(0, S, zero_body, 0)

    def part_body(p, _):
        pltpu.sync_copy(sums_hbm.at[p], stage_v)
        pltpu.sync_copy(cnts_hbm.at[p], cstage_v)

        def srow(s, _):
            for j in range(G):
                cs = pl.ds(j * L, L)
                attn_v[s, cs] = attn_v[s, cs] + stage_v[s, cs]
            ctot_v[s, :] = ctot_v[s, :] + cstage_v[s, :]
            return 0
        lax.fori_loop(0, S, srow, 0)
        return 0
    lax.fori_loop(0, NW, part_body, 0)

    def att_body(s, _):
        cn = jnp.maximum(ctot_v[s, :], 1.0)
        for j in range(G):
            cs = pl.ds(j * L, L)
            m = attn_v[s, cs] / cn
            attn_v[s, cs] = 1.0 / (1.0 + jnp.exp(-m))
        return 0
    lax.fori_loop(0, S, att_body, 0)

    def block_body(b, _):
        row0 = base + b * B
        pltpu.sync_copy(feats_hbm.at[pl.ds(row0, B), :], rows_v)
        pltpu.sync_copy(idx_hbm.at[pl.ds(row0 // L, CH), :], idxb_v)

        def chunk_body(c, _):
            iv = idxb_v[c, :]
            smin = jnp.min(iv)
            smax = jnp.max(iv)
            r0 = c * L

            @pl.when(smin == smax)
            def _fast():
                for j in range(G):
                    cs = pl.ds(j * L, L)
                    a = attn_v[smin, cs]
                    for r in range(L):
                        rows_v[r0 + r, cs] = rows_v[r0 + r, cs] * a

            @pl.when(smin != smax)
            def _slow():
                def row_body(r, _):
                    sr = _row_seg(iv, r)
                    for j in range(G):
                        cs = pl.ds(j * L, L)
                        rows_v[r0 + r, cs] = rows_v[r0 + r, cs] * attn_v[sr, cs]
                    return 0
                lax.fori_loop(0, L, row_body, 0)

            return 0
        lax.fori_loop(0, CH, chunk_body, 0)

        pltpu.sync_copy(rows_v, out_hbm.at[pl.ds(row0, B), :])
        return 0
    lax.fori_loop(0, NB, block_body, 0)


@jax.jit
def kernel(feats, batch_idx):
    idx2d = batch_idx.astype(jnp.int32).reshape(N // L, L)
    sums, cnts = _seg_partials(feats, idx2d)
    return _apply(feats, idx2d, sums, cnts)


# speedup vs baseline: 3.9646x; 3.9646x over previous
import functools

import jax
import jax.numpy as jnp
from jax import lax
from jax.experimental import pallas as pl
from jax.experimental.pallas import tpu as pltpu
from jax.experimental.pallas import tpu_sc as plsc

N = 320000
D = 128
S = 64
NC = 2
NS = 16
NW = NC * NS
R = N // NW
B = 400
NB = R // B
L = 16
CH = B // L
G = D // L

_mesh = plsc.VectorSubcoreMesh(
    core_axis_name="c", subcore_axis_name="s", num_cores=NC, num_subcores=NS)


def _worker_id():
    return lax.axis_index("s") * NC + lax.axis_index("c")


def _row_seg(iv, r):
    return iv[r]


@functools.partial(
    pl.kernel,
    out_type=[
        jax.ShapeDtypeStruct((NW, S, D), jnp.float32),
        jax.ShapeDtypeStruct((NW, S, L), jnp.float32),
    ],
    mesh=_mesh,
    scratch_types=[
        pltpu.VMEM((B, D), jnp.float32),
        pltpu.VMEM((B,), jnp.int32),
        pltpu.VMEM((S, D), jnp.float32),
        pltpu.VMEM((S, L), jnp.float32),
    ],
)
def _seg_partials(feats_hbm, idx_hbm, sums_hbm, cnts_hbm,
                  rows_v, idxb_v, acc_v, cnt_v):
    wid = _worker_id()
    base = wid * R
    zf = jnp.zeros((L,), jnp.float32)

    def zero_body(s, _):
        for j in range(G):
            acc_v[s, pl.ds(j * L, L)] = zf
        cnt_v[s, :] = zf
        return 0
    lax.fori_loop(0, S, zero_body, 0)

    def block_body(b, _):
        row0 = base + b * B
        pltpu.sync_copy(feats_hbm.at[pl.ds(row0, B), :], rows_v)
        pltpu.sync_copy(idx_hbm.at[pl.ds(row0, B)], idxb_v)

        def chunk_body(c, _):
            iv = idxb_v[pl.ds(pl.multiple_of(c * L, L), L)]
            smin = iv[0]
            smax = iv[L - 1]
            r0 = c * L

            @pl.when(smin == smax)
            def _fast():
                for j in range(G):
                    cs = pl.ds(j * L, L)
                    sacc = rows_v[r0, cs]
                    for r in range(1, L):
                        sacc = sacc + rows_v[r0 + r, cs]
                    acc_v[smin, cs] = acc_v[smin, cs] + sacc
                cnt_v[smin, :] = cnt_v[smin, :] + jnp.full((L,), 16.0, jnp.float32)

            @pl.when(smin != smax)
            def _slow():
                for r in range(L):
                    sr = iv[r]
                    for j in range(G):
                        cs = pl.ds(j * L, L)
                        acc_v[sr, cs] = acc_v[sr, cs] + rows_v[r0 + r, cs]
                    cnt_v[sr, :] = cnt_v[sr, :] + jnp.full((L,), 1.0, jnp.float32)

            return 0
        lax.fori_loop(0, CH, chunk_body, 0)
        return 0
    lax.fori_loop(0, NB, block_body, 0)

    pltpu.sync_copy(acc_v, sums_hbm.at[wid])
    pltpu.sync_copy(cnt_v, cnts_hbm.at[wid])


@functools.partial(
    pl.kernel,
    out_type=jax.ShapeDtypeStruct((N, D), jnp.float32),
    mesh=_mesh,
    scratch_types=[
        pltpu.VMEM((B, D), jnp.float32),
        pltpu.VMEM((B,), jnp.int32),
        pltpu.VMEM((S, D), jnp.float32),
        pltpu.VMEM((S, D), jnp.float32),
        pltpu.VMEM((S, L), jnp.float32),
        pltpu.VMEM((S, L), jnp.float32),
    ],
)
def _apply(feats_hbm, idx_hbm, sums_hbm, cnts_hbm, out_hbm,
           rows_v, idxb_v, attn_v, stage_v, ctot_v, cstage_v):
    wid = _worker_id()
    base = wid * R
    zf = jnp.zeros((L,), jnp.float32)

    def zero_body(s, _):
        for j in range(G):
            attn_v[s, pl.ds(j * L, L)] = zf
        ctot_v[s, :] = zf
        return 0
    lax.fori_loop(0, S, zero_body, 0)

    def part_body(p, _):
        pltpu.sync_copy(sums_hbm.at[p], stage_v)
        pltpu.sync_copy(cnts_hbm.at[p], cstage_v)

        def srow(s, _):
            for j in range(G):
                cs = pl.ds(j * L, L)
                attn_v[s, cs] = attn_v[s, cs] + stage_v[s, cs]
            ctot_v[s, :] = ctot_v[s, :] + cstage_v[s, :]
            return 0
        lax.fori_loop(0, S, srow, 0)
        return 0
    lax.fori_loop(0, NW, part_body, 0)

    def att_body(s, _):
        cn = jnp.maximum(ctot_v[s, :], 1.0)
        for j in range(G):
            cs = pl.ds(j * L, L)
            m = attn_v[s, cs] / cn
            attn_v[s, cs] = 1.0 / (1.0 + jnp.exp(-m))
        return 0
    lax.fori_loop(0, S, att_body, 0)

    def block_body(b, _):
        row0 = base + b * B
        pltpu.sync_copy(feats_hbm.at[pl.ds(row0, B), :], rows_v)
        pltpu.sync_copy(idx_hbm.at[pl.ds(row0, B)], idxb_v)

        def chunk_body(c, _):
            iv = idxb_v[pl.ds(pl.multiple_of(c * L, L), L)]
            smin = iv[0]
            smax = iv[L - 1]
            r0 = c * L

            @pl.when(smin == smax)
            def _fast():
                for j in range(G):
                    cs = pl.ds(j * L, L)
                    a = attn_v[smin, cs]
                    for r in range(L):
                        rows_v[r0 + r, cs] = rows_v[r0 + r, cs] * a

            @pl.when(smin != smax)
            def _slow():
                for r in range(L):
                    sr = iv[r]
                    for j in range(G):
                        cs = pl.ds(j * L, L)
                        rows_v[r0 + r, cs] = rows_v[r0 + r, cs] * attn_v[sr, cs]

            return 0
        lax.fori_loop(0, CH, chunk_body, 0)

        pltpu.sync_copy(rows_v, out_hbm.at[pl.ds(row0, B), :])
        return 0
    lax.fori_loop(0, NB, block_body, 0)


@jax.jit
def kernel(feats, batch_idx):
    idx32 = batch_idx.astype(jnp.int32)
    sums, cnts = _seg_partials(feats, idx32)
    return _apply(feats, idx32, sums, cnts)


# speedup vs baseline: 5.9978x; 1.5128x over previous
import functools

import jax
import jax.numpy as jnp
from jax import lax
from jax.experimental import pallas as pl
from jax.experimental.pallas import tpu as pltpu
from jax.experimental.pallas import tpu_sc as plsc

N = 320000
D = 128
S = 64
NC = 2
NS = 16
NW = NC * NS
R = N // NW
L = 16
G = D // L

B1 = 400
NB1 = R // B1
CH1 = B1 // L

B2 = 80
NB2 = R // B2
CH2 = B2 // L

SG = S // 8

_mesh = plsc.VectorSubcoreMesh(
    core_axis_name="c", subcore_axis_name="s", num_cores=NC, num_subcores=NS)


def _worker_id():
    return lax.axis_index("s") * NC + lax.axis_index("c")


@functools.partial(
    pl.kernel,
    out_type=[
        jax.ShapeDtypeStruct((NW, S, D), jnp.float32),
        jax.ShapeDtypeStruct((NW, S, L), jnp.float32),
    ],
    mesh=_mesh,
    scratch_types=[
        [pltpu.VMEM((B1, D), jnp.float32)] * 2,
        [pltpu.VMEM((B1,), jnp.int32)] * 2,
        pltpu.VMEM((S, D), jnp.float32),
        pltpu.VMEM((S, L), jnp.float32),
        [pltpu.SemaphoreType.DMA] * 2,
        [pltpu.SemaphoreType.DMA] * 2,
    ],
)
def _seg_partials(feats_hbm, idx_hbm, sums_hbm, cnts_hbm,
                  rows, idxb, acc_v, cnt_v, sr, si):
    wid = _worker_id()
    base = wid * R
    zf = jnp.zeros((L,), jnp.float32)

    def zero_body(s, _):
        for j in range(G):
            acc_v[s, pl.ds(j * L, L)] = zf
        cnt_v[s, :] = zf
        return 0
    lax.fori_loop(0, S, zero_body, 0)

    def start_in(b, k):
        row0 = base + b * B1
        pltpu.async_copy(feats_hbm.at[pl.ds(row0, B1), :], rows[k], sr[k])
        pltpu.async_copy(idx_hbm.at[pl.ds(row0, B1)], idxb[k], si[k])

    def wait_in(k):
        pltpu.make_async_copy(feats_hbm.at[pl.ds(0, B1), :], rows[k], sr[k]).wait()
        pltpu.make_async_copy(idx_hbm.at[pl.ds(0, B1)], idxb[k], si[k]).wait()

    def compute(k):
        rows_v = rows[k]
        idxb_v = idxb[k]

        def chunk_body(c, _):
            iv = idxb_v[pl.ds(pl.multiple_of(c * L, L), L)]
            smin = iv[0]
            smax = iv[L - 1]
            r0 = c * L

            @pl.when(smin == smax)
            def _fast():
                for j in range(G):
                    cs = pl.ds(j * L, L)
                    sacc = rows_v[r0, cs]
                    for r in range(1, L):
                        sacc = sacc + rows_v[r0 + r, cs]
                    acc_v[smin, cs] = acc_v[smin, cs] + sacc
                cnt_v[smin, :] = cnt_v[smin, :] + jnp.full((L,), 16.0, jnp.float32)

            @pl.when(smin != smax)
            def _slow():
                for r in range(L):
                    sr_ = iv[r]
                    for j in range(G):
                        cs = pl.ds(j * L, L)
                        acc_v[sr_, cs] = acc_v[sr_, cs] + rows_v[r0 + r, cs]
                    cnt_v[sr_, :] = cnt_v[sr_, :] + jnp.full((L,), 1.0, jnp.float32)

            return 0
        lax.fori_loop(0, CH1, chunk_body, 0)

    start_in(0, 0)

    def outer(bb, _):
        for k in range(2):
            b = 2 * bb + k
            wait_in(k)
            start_in(b + 1, k ^ 1)
            compute(k)
        return 0
    lax.fori_loop(0, (NB1 - 1) // 2, outer, 0)

    wait_in(0)
    compute(0)

    pltpu.sync_copy(acc_v, sums_hbm.at[wid])
    pltpu.sync_copy(cnt_v, cnts_hbm.at[wid])


@functools.partial(
    pl.kernel,
    out_type=jax.ShapeDtypeStruct((S, D), jnp.float32),
    mesh=_mesh,
    scratch_types=[
        pltpu.VMEM((NW, SG, D), jnp.float32),
        pltpu.VMEM((NW, SG, L), jnp.float32),
        pltpu.VMEM((SG, D), jnp.float32),
        pltpu.VMEM((SG, L), jnp.float32),
    ],
)
def _attention(sums_hbm, cnts_hbm, attn_hbm, stage_v, cstage_v, acc_v, ctot_v):
    wid = _worker_id()

    @pl.when(wid < S // SG)
    def _active():
        seg0 = wid * SG
        pltpu.sync_copy(sums_hbm.at[:, pl.ds(seg0, SG), :], stage_v)
        pltpu.sync_copy(cnts_hbm.at[:, pl.ds(seg0, SG), :], cstage_v)

        zf = jnp.zeros((L,), jnp.float32)
        for s in range(SG):
            for j in range(G):
                acc_v[s, pl.ds(j * L, L)] = zf
            ctot_v[s, :] = zf

        def red_body(p, _):
            for s in range(SG):
                for j in range(G):
                    cs = pl.ds(j * L, L)
                    acc_v[s, cs] = acc_v[s, cs] + stage_v[p, s, cs]
                ctot_v[s, :] = ctot_v[s, :] + cstage_v[p, s, :]
            return 0
        lax.fori_loop(0, NW, red_body, 0)

        for s in range(SG):
            cn = jnp.maximum(ctot_v[s, :], 1.0)
            for j in range(G):
                cs = pl.ds(j * L, L)
                m = acc_v[s, cs] / cn
                acc_v[s, cs] = 1.0 / (1.0 + jnp.exp(-m))

        pltpu.sync_copy(acc_v, attn_hbm.at[pl.ds(seg0, SG), :])


@functools.partial(
    pl.kernel,
    out_type=jax.ShapeDtypeStruct((N, D), jnp.float32),
    mesh=_mesh,
    scratch_types=[
        [pltpu.VMEM((B2, D), jnp.float32)] * 2,
        [pltpu.VMEM((B2, D), jnp.float32)] * 2,
        [pltpu.VMEM((B2,), jnp.int32)] * 2,
        pltpu.VMEM((S, D), jnp.float32),
        [pltpu.SemaphoreType.DMA] * 2,
        [pltpu.SemaphoreType.DMA] * 2,
        [pltpu.SemaphoreType.DMA] * 2,
    ],
)
def _apply(feats_hbm, idx_hbm, attn_hbm, out_hbm,
           rin, rout, idxb, attn_v, sr, si, so):
    wid = _worker_id()
    base = wid * R

    pltpu.sync_copy(attn_hbm, attn_v)

    def start_in(b, k):
        row0 = base + b * B2
        pltpu.async_copy(feats_hbm.at[pl.ds(row0, B2), :], rin[k], sr[k])
        pltpu.async_copy(idx_hbm.at[pl.ds(row0, B2)], idxb[k], si[k])

    def wait_in(k):
        pltpu.make_async_copy(feats_hbm.at[pl.ds(0, B2), :], rin[k], sr[k]).wait()
        pltpu.make_async_copy(idx_hbm.at[pl.ds(0, B2)], idxb[k], si[k]).wait()

    def start_out(b, k):
        row0 = base + b * B2
        pltpu.async_copy(rout[k], out_hbm.at[pl.ds(row0, B2), :], so[k])

    def wait_out(k):
        pltpu.make_async_copy(rout[k], out_hbm.at[pl.ds(0, B2), :], so[k]).wait()

    def compute(k):
        rin_v = rin[k]
        rout_v = rout[k]
        idxb_v = idxb[k]

        def chunk_body(c, _):
            iv = idxb_v[pl.ds(pl.multiple_of(c * L, L), L)]
            smin = iv[0]
            smax = iv[L - 1]
            r0 = c * L

            @pl.when(smin == smax)
            def _fast():
                for j in range(G):
                    cs = pl.ds(j * L, L)
                    a = attn_v[smin, cs]
                    for r in range(L):
                        rout_v[r0 + r, cs] = rin_v[r0 + r, cs] * a

            @pl.when(smin != smax)
            def _slow():
                for r in range(L):
                    sr_ = iv[r]
                    for j in range(G):
                        cs = pl.ds(j * L, L)
                        rout_v[r0 + r, cs] = rin_v[r0 + r, cs] * attn_v[sr_, cs]

            return 0
        lax.fori_loop(0, CH2, chunk_body, 0)

    start_in(0, 0)
    wait_in(0)
    start_in(1, 1)
    compute(0)
    start_out(0, 0)
    wait_in(1)
    start_in(2, 0)
    compute(1)
    start_out(1, 1)

    def outer(bb, _):
        for k in range(2):
            b = 2 * bb + k
            wait_in(k)
            start_in(b + 1, k ^ 1)
            wait_out(k)
            compute(k)
            start_out(b, k)
        return 0
    lax.fori_loop(1, (NB2 - 1) // 2, outer, 0)

    wait_in(0)
    wait_out(0)
    compute(0)
    start_out(NB2 - 1, 0)

    wait_out(1)
    wait_out(0)


@jax.jit
def kernel(feats, batch_idx):
    idx32 = batch_idx.astype(jnp.int32)
    sums, cnts = _seg_partials(feats, idx32)
    attn = _attention(sums, cnts)
    return _apply(feats, idx32, attn)
